# Initial kernel scaffold; baseline (speedup 1.0000x reference)
import functools

import jax
import jax.numpy as jnp
from jax import lax
from jax.experimental import pallas as pl
from jax.experimental.pallas import tpu as pltpu
from jax.experimental.pallas import tpu_sc as plsc

_D = 64
_NC = 2
_NS = 16
_NW = _NC * _NS
_IW = 128
_C = 1024
_SUB = _C // _IW


@functools.lru_cache(maxsize=None)
def _build(n_flat: int):
    b_per_w = n_flat // _NW
    n_chunks = b_per_w // _C
    mesh = plsc.VectorSubcoreMesh(core_axis_name="c", subcore_axis_name="s")

    @functools.partial(
        pl.kernel,
        mesh=mesh,
        out_type=jax.ShapeDtypeStruct((n_flat, _D), jnp.float32),
        scratch_types=[
            pltpu.VMEM((_SUB, _IW), jnp.int32),
            pltpu.VMEM((_C, _D), jnp.float32),
            pltpu.SemaphoreType.DMA,
        ],
    )
    def k(idx_hbm, table_hbm, out_hbm, idx_v, rows_v, sem):
        wid = lax.axis_index("s") * _NC + lax.axis_index("c")
        base = wid * b_per_w

        def body(g, carry):
            off = base + g * _C
            pltpu.sync_copy(idx_hbm.at[pl.ds(off // _IW, _SUB)], idx_v)
            copies = [
                pltpu.async_copy(
                    table_hbm.at[idx_v.at[j]],
                    rows_v.at[pl.ds(j * _IW, _IW)],
                    sem,
                )
                for j in range(_SUB)
            ]
            for c in copies:
                c.wait()
            pltpu.sync_copy(rows_v, out_hbm.at[pl.ds(off, _C)])
            return carry

        lax.fori_loop(0, n_chunks, body, 0)

    return k


def kernel(token_ids, table):
    b, h = token_ids.shape
    flat = token_ids.reshape(-1).astype(jnp.int32)
    n = b * h
    step = _NW * _C
    n_pad = -(-n // step) * step
    if n_pad != n:
        flat = jnp.concatenate([flat, jnp.zeros((n_pad - n,), jnp.int32)])
    idx2d = flat.reshape(-1, _IW)
    out = _build(n_pad)(idx2d, table)
    return out[:n].reshape(b, h, _D)


# speedup vs baseline: 1.8548x; 1.8548x over previous
import functools

import jax
import jax.numpy as jnp
from jax import lax
from jax.experimental import pallas as pl
from jax.experimental.pallas import tpu as pltpu
from jax.experimental.pallas import tpu_sc as plsc

_D = 64
_NC = 2
_NS = 16
_NW = _NC * _NS
_IW = 128
_C = 1024
_SUB = _C // _IW


@functools.lru_cache(maxsize=None)
def _build(n_flat: int):
    b_per_w = n_flat // _NW
    n_chunks = b_per_w // _C
    mesh = plsc.VectorSubcoreMesh(core_axis_name="c", subcore_axis_name="s")

    @functools.partial(
        pl.kernel,
        mesh=mesh,
        out_type=jax.ShapeDtypeStruct((n_flat, _D), jnp.float32),
        scratch_types=[
            pltpu.VMEM((_SUB, _IW), jnp.int32),
            pltpu.VMEM((_C, _D), jnp.float32),
            pltpu.SemaphoreType.DMA,
        ],
        compiler_params=pltpu.CompilerParams(use_tc_tiling_on_sc=False),
    )
    def k(idx_hbm, table_hbm, out_hbm, idx_v, rows_v, sem):
        wid = lax.axis_index("s") * _NC + lax.axis_index("c")
        base = wid * b_per_w

        def body(g, carry):
            off = pl.multiple_of(base + g * _C, _C)
            row = pl.multiple_of(off // _IW, _SUB)
            pltpu.sync_copy(idx_hbm.at[pl.ds(row, _SUB)], idx_v)
            copies = [
                pltpu.async_copy(
                    table_hbm.at[idx_v.at[j]],
                    rows_v.at[pl.ds(j * _IW, _IW)],
                    sem,
                )
                for j in range(_SUB)
            ]
            for c in copies:
                c.wait()
            pltpu.sync_copy(rows_v, out_hbm.at[pl.ds(off, _C)])
            return carry

        lax.fori_loop(0, n_chunks, body, 0)

    return k


def kernel(token_ids, table):
    b, h = token_ids.shape
    flat = token_ids.reshape(-1).astype(jnp.int32)
    n = b * h
    step = _NW * _C
    n_pad = -(-n // step) * step
    if n_pad != n:
        flat = jnp.concatenate([flat, jnp.zeros((n_pad - n,), jnp.int32)])
    idx2d = flat.reshape(-1, _IW)
    out = _build(n_pad)(idx2d, table)
    return out[:n].reshape(b, h, _D)


# speedup vs baseline: 1.8706x; 1.0085x over previous
import functools

import jax
import jax.numpy as jnp
from jax import lax
from jax.experimental import pallas as pl
from jax.experimental.pallas import tpu as pltpu
from jax.experimental.pallas import tpu_sc as plsc

_D = 64
_NC = 2
_NS = 16
_NW = _NC * _NS
_IW = 128
_C = 512
_SUB = _C // _IW
_NB = 2


@functools.lru_cache(maxsize=None)
def _build(n_flat: int):
    b_per_w = n_flat // _NW
    n_chunks = b_per_w // _C
    assert n_chunks % _NB == 0
    mesh = plsc.VectorSubcoreMesh(core_axis_name="c", subcore_axis_name="s")

    @functools.partial(
        pl.kernel,
        mesh=mesh,
        out_type=jax.ShapeDtypeStruct((n_flat, _D), jnp.float32),
        scratch_types=[
            pltpu.VMEM((_NB, _SUB, _IW), jnp.int32),
            pltpu.VMEM((_NB, _C, _D), jnp.float32),
            [pltpu.SemaphoreType.DMA] * _NB,
            [pltpu.SemaphoreType.DMA] * _NB,
            [pltpu.SemaphoreType.DMA] * _NB,
        ],
        compiler_params=pltpu.CompilerParams(use_tc_tiling_on_sc=False),
    )
    def k(idx_hbm, table_hbm, out_hbm, idx_v, rows_v, sem_i, sem_g, sem_o):
        wid = lax.axis_index("s") * _NC + lax.axis_index("c")
        base = wid * b_per_w
        base_r = base // _IW

        def idx_copy(b, g):
            row = pl.multiple_of(base_r + g * _SUB, _SUB)
            return pltpu.make_async_copy(
                idx_hbm.at[pl.ds(row, _SUB)], idx_v.at[b], sem_i[b])

        def out_copy(b, g):
            off = pl.multiple_of(base + g * _C, _C)
            return pltpu.make_async_copy(
                rows_v.at[b], out_hbm.at[pl.ds(off, _C)], sem_o[b])

        for b in range(_NB):
            idx_copy(b, b).start()

        @pl.loop(0, n_chunks, step=_NB)
        def _(i):
            for b in range(_NB):
                g = i + b
                idx_copy(b, g).wait()

                @pl.when(g >= _NB)
                def _():
                    out_copy(b, g - _NB).wait()

                gathers = [
                    pltpu.async_copy(
                        table_hbm.at[idx_v.at[b, j]],
                        rows_v.at[b, pl.ds(j * _IW, _IW)],
                        sem_g[b],
                    )
                    for j in range(_SUB)
                ]
                for c in gathers:
                    c.wait()
                out_copy(b, g).start()

                @pl.when(g + _NB < n_chunks)
                def _():
                    idx_copy(b, g + _NB).start()

        for b in range(_NB):
            out_copy(b, n_chunks - _NB + b).wait()

    return k


def kernel(token_ids, table):
    b, h = token_ids.shape
    flat = token_ids.reshape(-1).astype(jnp.int32)
    n = b * h
    step = _NW * _C * _NB
    n_pad = -(-n // step) * step
    if n_pad != n:
        flat = jnp.concatenate([flat, jnp.zeros((n_pad - n,), jnp.int32)])
    idx2d = flat.reshape(-1, _IW)
    out = _build(n_pad)(idx2d, table)
    return out[:n].reshape(b, h, _D)


# speedup vs baseline: 1.8777x; 1.0038x over previous
import functools

import jax
import jax.numpy as jnp
from jax import lax
from jax.experimental import pallas as pl
from jax.experimental.pallas import tpu as pltpu
from jax.experimental.pallas import tpu_sc as plsc

_D = 64
_NC = 2
_NS = 16
_NW = _NC * _NS
_IW = 128
_C = 256
_SUB = _C // _IW
_NB = 5
_A = 3


@functools.lru_cache(maxsize=None)
def _build(n_flat: int):
    b_per_w = n_flat // _NW
    n_chunks = b_per_w // _C
    assert n_chunks % _NB == 0 and _A < _NB
    mesh = plsc.VectorSubcoreMesh(core_axis_name="c", subcore_axis_name="s")

    @functools.partial(
        pl.kernel,
        mesh=mesh,
        out_type=jax.ShapeDtypeStruct((n_flat, _D), jnp.float32),
        scratch_types=[
            pltpu.VMEM((_NB, _SUB, _IW), jnp.int32),
            pltpu.VMEM((_NB, _C, _D), jnp.float32),
            [pltpu.SemaphoreType.DMA] * _NB,
            [pltpu.SemaphoreType.DMA] * _NB,
            [pltpu.SemaphoreType.DMA] * _NB,
        ],
        compiler_params=pltpu.CompilerParams(use_tc_tiling_on_sc=False),
    )
    def k(idx_hbm, table_hbm, out_hbm, idx_v, rows_v, sem_i, sem_g, sem_o):
        wid = lax.axis_index("s") * _NC + lax.axis_index("c")
        base = wid * b_per_w
        base_r = base // _IW

        def idx_copy(b, g):
            row = pl.multiple_of(base_r + g * _SUB, _SUB)
            return pltpu.make_async_copy(
                idx_hbm.at[pl.ds(row, _SUB)], idx_v.at[b], sem_i[b])

        def out_copy(b, g):
            off = pl.multiple_of(base + g * _C, _C)
            return pltpu.make_async_copy(
                rows_v.at[b], out_hbm.at[pl.ds(off, _C)], sem_o[b])

        def fire_gathers(b):
            for j in range(_SUB):
                pltpu.async_copy(
                    table_hbm.at[idx_v.at[b, j]],
                    rows_v.at[b, pl.ds(j * _IW, _IW)],
                    sem_g[b],
                )

        def drain_gathers(b):
            for j in range(_SUB):
                pltpu.make_async_copy(
                    table_hbm.at[idx_v.at[b, j]],
                    rows_v.at[b, pl.ds(j * _IW, _IW)],
                    sem_g[b],
                ).wait()

        for b in range(_NB):
            idx_copy(b, b).start()
        for g in range(_A):
            idx_copy(g, g).wait()
            fire_gathers(g)

        @pl.loop(0, n_chunks, step=_NB)
        def _(i):
            for b in range(_NB):
                g = i + b
                ba = (b + _A) % _NB

                @pl.when(g + _A < n_chunks)
                def _():
                    idx_copy(ba, g + _A).wait()

                    @pl.when(g + _A >= _NB)
                    def _():
                        out_copy(ba, g + _A - _NB).wait()

                    fire_gathers(ba)

                drain_gathers(b)
                out_copy(b, g).start()

                @pl.when(g + _NB < n_chunks)
                def _():
                    idx_copy(b, g + _NB).start()

        for b in range(_NB):
            out_copy(b, n_chunks - _NB + b).wait()

    return k


def kernel(token_ids, table):
    b, h = token_ids.shape
    flat = token_ids.reshape(-1).astype(jnp.int32)
    n = b * h
    step = _NW * _C * _NB
    n_pad = -(-n // step) * step
    if n_pad != n:
        flat = jnp.concatenate([flat, jnp.zeros((n_pad - n,), jnp.int32)])
    idx2d = flat.reshape(-1, _IW)
    out = _build(n_pad)(idx2d, table)
    return out[:n].reshape(b, h, _D)
